# Initial kernel scaffold; baseline (speedup 1.0000x reference)
import jax
import jax.numpy as jnp
from jax.experimental import pallas as pl


def kernel(input, inds, w, b):
    raise NotImplementedError("write your pallas kernel here")


# speedup vs baseline: 2.6488x; 2.6488x over previous
import jax
import jax.numpy as jnp
from jax.experimental import pallas as pl
from jax.experimental.pallas import tpu as pltpu

CLASSES = 64
IN_F = 128
OUT_F = 128
N = 4096


def _body(inds_ref, x_ref, w_ref, b_ref, out_ref):
    e = pl.program_id(0)

    @pl.when(e == 0)
    def _init():
        out_ref[...] = jnp.zeros_like(out_ref)

    mask = inds_ref[...] == e
    xm = jnp.where(mask, x_ref[...], 0.0)
    contrib = jnp.dot(xm, w_ref[0], preferred_element_type=jnp.float32)
    bias = jnp.where(mask, b_ref[0], 0.0)
    out_ref[...] += contrib + bias


def kernel(input, inds, w, b):
    inds2d = inds.astype(jnp.int32).reshape(N, 1)
    return pl.pallas_call(
        _body,
        grid=(CLASSES,),
        in_specs=[
            pl.BlockSpec((N, 1), lambda e: (0, 0)),
            pl.BlockSpec((N, IN_F), lambda e: (0, 0)),
            pl.BlockSpec((1, IN_F, OUT_F), lambda e: (e, 0, 0)),
            pl.BlockSpec((1, 1, OUT_F), lambda e: (e, 0, 0)),
        ],
        out_specs=pl.BlockSpec((N, OUT_F), lambda e: (0, 0)),
        out_shape=jax.ShapeDtypeStruct((N, OUT_F), jnp.float32),
    )(inds2d, input, w, b)
